# Initial kernel scaffold; baseline (speedup 1.0000x reference)
import jax
import jax.numpy as jnp
from jax.experimental import pallas as pl


def kernel(install, install_ecc, app_table):
    raise NotImplementedError("write your pallas kernel here")


# speedup vs baseline: 2.2073x; 2.2073x over previous
import functools

import jax
import jax.numpy as jnp
from jax import lax
from jax.experimental import pallas as pl
from jax.experimental.pallas import tpu as pltpu
from jax.experimental.pallas import tpu_sc as plsc

APP_DIM = 32
B = 4096
L_INSTALL = 200
L_ECC = 50
NC = 2
NS = 16
NW = NC * NS
BPW = B // NW
INST_CHUNK = 100


def _body(install_hbm, ecc_hbm, table_hbm, out_hbm,
          idx_i_v, idx_e_v, rows_i_v, rows_e_v, out_v, sem):
    wid = lax.axis_index("s") * NC + lax.axis_index("c")
    base = wid * BPW

    pltpu.sync_copy(install_hbm.at[pl.ds(base, BPW)], idx_i_v)
    pltpu.sync_copy(ecc_hbm.at[pl.ds(base, BPW)], idx_e_v)

    zero = jnp.zeros((16,), jnp.float32)

    def sample(i, carry):
        c0 = pltpu.async_copy(
            table_hbm.at[idx_i_v.at[i, 0]], rows_i_v.at[pl.ds(0, INST_CHUNK)], sem)
        c1 = pltpu.async_copy(
            table_hbm.at[idx_i_v.at[i, 1]],
            rows_i_v.at[pl.ds(INST_CHUNK, INST_CHUNK)], sem)
        c2 = pltpu.async_copy(table_hbm.at[idx_e_v.at[i]], rows_e_v, sem)
        c0.wait()
        c1.wait()
        c2.wait()

        def red_i(j, acc):
            a0, a1 = acc
            return (a0 + rows_i_v[j, pl.ds(0, 16)],
                    a1 + rows_i_v[j, pl.ds(16, 16)])

        s0, s1 = lax.fori_loop(0, L_INSTALL, red_i, (zero, zero))
        out_v[i, pl.ds(0, 16)] = s0 * (1.0 / L_INSTALL)
        out_v[i, pl.ds(16, 16)] = s1 * (1.0 / L_INSTALL)

        def red_e(j, acc):
            a0, a1 = acc
            return (a0 + rows_e_v[j, pl.ds(0, 16)],
                    a1 + rows_e_v[j, pl.ds(16, 16)])

        e0, e1 = lax.fori_loop(0, L_ECC, red_e, (zero, zero))
        out_v[i, pl.ds(32, 16)] = e0 * (1.0 / L_ECC)
        out_v[i, pl.ds(48, 16)] = e1 * (1.0 / L_ECC)
        return carry

    lax.fori_loop(0, BPW, sample, 0)
    pltpu.sync_copy(out_v, out_hbm.at[pl.ds(base, BPW)])


@jax.jit
def kernel(install, install_ecc, app_table):
    install3 = install.astype(jnp.int32).reshape(B, 2, INST_CHUNK)
    ecc = install_ecc.astype(jnp.int32)
    mesh = plsc.VectorSubcoreMesh(core_axis_name="c", subcore_axis_name="s")
    run = pl.kernel(
        _body,
        mesh=mesh,
        out_type=jax.ShapeDtypeStruct((B, 2 * APP_DIM), jnp.float32),
        scratch_types=[
            pltpu.VMEM((BPW, 2, INST_CHUNK), jnp.int32),
            pltpu.VMEM((BPW, L_ECC), jnp.int32),
            pltpu.VMEM((L_INSTALL, APP_DIM), jnp.float32),
            pltpu.VMEM((L_ECC, APP_DIM), jnp.float32),
            pltpu.VMEM((BPW, 2 * APP_DIM), jnp.float32),
            pltpu.SemaphoreType.DMA,
        ],
        compiler_params=pltpu.CompilerParams(use_tc_tiling_on_sc=False),
    )
    return run(install3, ecc, app_table)


# speedup vs baseline: 2.9258x; 1.3255x over previous
import jax
import jax.numpy as jnp
from jax import lax
from jax.experimental import pallas as pl
from jax.experimental.pallas import tpu as pltpu
from jax.experimental.pallas import tpu_sc as plsc

APP_DIM = 32
B = 4096
L_INSTALL = 200
L_ECC = 50
NC = 2
NS = 16
NW = NC * NS
BPW = B // NW
INST_CHUNK = 100
NBUF = 4
UNROLL = 5


def _body(install_hbm, ecc_hbm, table_hbm, out_hbm,
          idx_i_v, idx_e_v, rows_i_v, rows_e_v, out_v, *sems):
    wid = lax.axis_index("s") * NC + lax.axis_index("c")
    base = wid * BPW

    pltpu.sync_copy(install_hbm.at[pl.ds(base, BPW)], idx_i_v)
    pltpu.sync_copy(ecc_hbm.at[pl.ds(base, BPW)], idx_e_v)

    def fire(i, k):
        pltpu.async_copy(table_hbm.at[idx_i_v.at[i, 0]],
                         rows_i_v.at[k, pl.ds(0, INST_CHUNK)], sems[k])
        pltpu.async_copy(table_hbm.at[idx_i_v.at[i, 1]],
                         rows_i_v.at[k, pl.ds(INST_CHUNK, INST_CHUNK)], sems[k])
        pltpu.async_copy(table_hbm.at[idx_e_v.at[i]], rows_e_v.at[k], sems[k])

    def drain(i, k):
        pltpu.make_async_copy(table_hbm.at[idx_i_v.at[i, 0]],
                              rows_i_v.at[k, pl.ds(0, INST_CHUNK)],
                              sems[k]).wait()
        pltpu.make_async_copy(table_hbm.at[idx_i_v.at[i, 1]],
                              rows_i_v.at[k, pl.ds(INST_CHUNK, INST_CHUNK)],
                              sems[k]).wait()
        pltpu.make_async_copy(table_hbm.at[idx_e_v.at[i]], rows_e_v.at[k],
                              sems[k]).wait()

    zero = jnp.zeros((16,), jnp.float32)

    def reduce_to(i, k):
        def red_i(j, acc):
            accs = list(acc)
            for u in range(UNROLL):
                r = j * UNROLL + u
                accs[2 * u] = accs[2 * u] + rows_i_v[k, r, pl.ds(0, 16)]
                accs[2 * u + 1] = accs[2 * u + 1] + rows_i_v[k, r, pl.ds(16, 16)]
            return tuple(accs)

        acc = lax.fori_loop(0, L_INSTALL // UNROLL, red_i, (zero,) * (2 * UNROLL))
        s0 = acc[0] + acc[2] + acc[4] + acc[6] + acc[8]
        s1 = acc[1] + acc[3] + acc[5] + acc[7] + acc[9]
        out_v[i, pl.ds(0, 16)] = s0 * (1.0 / L_INSTALL)
        out_v[i, pl.ds(16, 16)] = s1 * (1.0 / L_INSTALL)

        def red_e(j, acc):
            accs = list(acc)
            for u in range(UNROLL):
                r = j * UNROLL + u
                accs[2 * u] = accs[2 * u] + rows_e_v[k, r, pl.ds(0, 16)]
                accs[2 * u + 1] = accs[2 * u + 1] + rows_e_v[k, r, pl.ds(16, 16)]
            return tuple(accs)

        acc = lax.fori_loop(0, L_ECC // UNROLL, red_e, (zero,) * (2 * UNROLL))
        e0 = acc[0] + acc[2] + acc[4] + acc[6] + acc[8]
        e1 = acc[1] + acc[3] + acc[5] + acc[7] + acc[9]
        out_v[i, pl.ds(32, 16)] = e0 * (1.0 / L_ECC)
        out_v[i, pl.ds(48, 16)] = e1 * (1.0 / L_ECC)

    for k in range(NBUF):
        fire(k, k)

    def group(g, carry):
        i0 = g * NBUF
        for k in range(NBUF):
            i = i0 + k
            drain(i, k)
            reduce_to(i, k)

            @pl.when(g < BPW // NBUF - 1)
            def _():
                fire(i + NBUF, k)

        return carry

    lax.fori_loop(0, BPW // NBUF, group, 0)
    pltpu.sync_copy(out_v, out_hbm.at[pl.ds(base, BPW)])


@jax.jit
def kernel(install, install_ecc, app_table):
    install3 = install.astype(jnp.int32).reshape(B, 2, INST_CHUNK)
    ecc = install_ecc.astype(jnp.int32)
    mesh = plsc.VectorSubcoreMesh(core_axis_name="c", subcore_axis_name="s")
    run = pl.kernel(
        _body,
        mesh=mesh,
        out_type=jax.ShapeDtypeStruct((B, 2 * APP_DIM), jnp.float32),
        scratch_types=[
            pltpu.VMEM((BPW, 2, INST_CHUNK), jnp.int32),
            pltpu.VMEM((BPW, L_ECC), jnp.int32),
            pltpu.VMEM((NBUF, L_INSTALL, APP_DIM), jnp.float32),
            pltpu.VMEM((NBUF, L_ECC, APP_DIM), jnp.float32),
            pltpu.VMEM((BPW, 2 * APP_DIM), jnp.float32),
        ] + [pltpu.SemaphoreType.DMA] * NBUF,
        compiler_params=pltpu.CompilerParams(use_tc_tiling_on_sc=False),
    )
    return run(install3, ecc, app_table)
